# Initial kernel scaffold; baseline (speedup 1.0000x reference)
import jax
import jax.numpy as jnp
from jax.experimental import pallas as pl


def kernel(boxes, scores, scores_bf_multiply):
    raise NotImplementedError("write your pallas kernel here")


# speedup vs baseline: 6.7963x; 6.7963x over previous
import jax
import jax.numpy as jnp
from jax.experimental import pallas as pl
from jax.experimental.pallas import tpu as pltpu

N = 5000
K = 80
IMG_H = 800.0
IMG_W = 1333.0
SCORE_THRESH = 0.05
NMS_THRESH = 0.5
PRE_NMS = 1000
TOPK = 100
PAD = 1024
NEG = -3.0e38


def _nms_body(col_ref, row_ref, cls_ref, ox1, oy1, ox2, oy2, osc, obf, ocls,
              iou_scr):
    c = col_ref[...]
    r = row_ref[...]
    off_c = c[:, 4:5]
    ax1 = c[:, 0:1] + off_c
    ay1 = c[:, 1:2] + off_c
    ax2 = c[:, 2:3] + off_c
    ay2 = c[:, 3:4] + off_c
    off_r = r[6:7, :]
    bx1 = r[0:1, :] + off_r
    by1 = r[1:2, :] + off_r
    bx2 = r[2:3, :] + off_r
    by2 = r[3:4, :] + off_r
    area_a = (c[:, 2:3] - c[:, 0:1]) * (c[:, 3:4] - c[:, 1:2])
    area_b = (r[2:3, :] - r[0:1, :]) * (r[3:4, :] - r[1:2, :])
    iw = jnp.clip(jnp.minimum(ax2, bx2) - jnp.maximum(ax1, bx1), 0.0, None)
    ih = jnp.clip(jnp.minimum(ay2, by2) - jnp.maximum(ay1, by1), 0.0, None)
    inter = iw * ih
    union = area_a + area_b - inter
    iou_scr[...] = jnp.where(union > 0.0, inter / union, 0.0)

    iota = jax.lax.broadcasted_iota(jnp.int32, (1, PAD), 1)
    s_row = r[4:5, :]
    keep0 = (s_row > NEG).astype(jnp.float32)

    def step(i, keep):
        row = iou_scr[pl.ds(i, 1), :]
        keep_i = jnp.max(jnp.where(iota == i, keep, 0.0))
        sup = (row > NMS_THRESH) & (iota > i) & (keep_i > 0.0)
        return jnp.where(sup, 0.0, keep)

    keep = jax.lax.fori_loop(0, PRE_NMS, step, keep0)
    final0 = jnp.where(keep > 0.0, s_row, NEG)

    bf_row = r[5:6, :]
    cls_row = cls_ref[...]
    iota_t = jax.lax.broadcasted_iota(jnp.int32, (1, 128), 1)
    zf = jnp.zeros((1, 128), jnp.float32)
    zi = jnp.zeros((1, 128), jnp.int32)

    def ext(t, carry):
        final, vx1, vy1, vx2, vy2, vsc, vbf, vcl = carry
        m = jnp.max(final)
        idx = jnp.min(jnp.where(final == m, iota, PAD))
        sel = iota == idx
        valid = m > NEG

        def pick(row):
            return jnp.sum(jnp.where(sel, row, 0.0))

        px1 = jnp.where(valid, pick(r[0:1, :]), 0.0)
        py1 = jnp.where(valid, pick(r[1:2, :]), 0.0)
        px2 = jnp.where(valid, pick(r[2:3, :]), 0.0)
        py2 = jnp.where(valid, pick(r[3:4, :]), 0.0)
        psc = jnp.where(valid, m, 0.0)
        pbf = jnp.where(valid, pick(bf_row), 0.0)
        pcl = jnp.where(valid, jnp.sum(jnp.where(sel, cls_row, 0)), -1)

        tt = iota_t == t
        carry2 = (jnp.where(sel, NEG, final),
                  jnp.where(tt, px1, vx1), jnp.where(tt, py1, vy1),
                  jnp.where(tt, px2, vx2), jnp.where(tt, py2, vy2),
                  jnp.where(tt, psc, vsc), jnp.where(tt, pbf, vbf),
                  jnp.where(tt, pcl, vcl))
        return carry2

    final, vx1, vy1, vx2, vy2, vsc, vbf, vcl = jax.lax.fori_loop(
        0, TOPK, ext, (final0, zf, zf, zf, zf, zf, zf, zi))
    ox1[...] = vx1
    oy1[...] = vy1
    ox2[...] = vx2
    oy2[...] = vy2
    osc[...] = vsc
    obf[...] = vbf
    ocls[...] = vcl


def kernel(boxes, scores, scores_bf_multiply):
    x1 = jnp.clip(boxes[:, 0], 0.0, IMG_W)
    y1 = jnp.clip(boxes[:, 1], 0.0, IMG_H)
    x2 = jnp.clip(boxes[:, 2], 0.0, IMG_W)
    y2 = jnp.clip(boxes[:, 3], 0.0, IMG_H)
    boxes_c = jnp.stack([x1, y1, x2, y2], axis=1)

    cls_scores = scores[:, :-1].reshape(-1)
    bf_scores = scores_bf_multiply[:, :-1].reshape(-1)
    masked = jnp.where(cls_scores > SCORE_THRESH, cls_scores, -jnp.inf)
    top_scores, order = jax.lax.top_k(masked, PRE_NMS)
    box_idx = order // K
    cls_idx = order % K
    cand = boxes_c[box_idx]
    top_bf = bf_scores[order]
    maxv = jnp.max(boxes_c)
    off = (maxv + 1.0) * cls_idx.astype(jnp.float32)

    pad = PAD - PRE_NMS
    cand_p = jnp.pad(cand, ((0, pad), (0, 0)))
    sc_p = jnp.pad(jnp.where(jnp.isfinite(top_scores), top_scores, NEG),
                   (0, pad), constant_values=NEG)
    bf_p = jnp.pad(top_bf, (0, pad))
    off_p = jnp.pad(off, (0, pad))
    cls_p = jnp.pad(cls_idx, (0, pad))

    col = jnp.concatenate(
        [cand_p, off_p[:, None], jnp.zeros((PAD, 3), jnp.float32)], axis=1)
    row = jnp.stack([cand_p[:, 0], cand_p[:, 1], cand_p[:, 2], cand_p[:, 3],
                     sc_p, bf_p, off_p, jnp.zeros((PAD,), jnp.float32)],
                    axis=0)
    cls_in = cls_p[None, :].astype(jnp.int32)

    of = jax.ShapeDtypeStruct((1, 128), jnp.float32)
    oi = jax.ShapeDtypeStruct((1, 128), jnp.int32)
    vx1, vy1, vx2, vy2, vsc, vbf, vcl = pl.pallas_call(
        _nms_body,
        out_shape=[of, of, of, of, of, of, oi],
        scratch_shapes=[pltpu.VMEM((PAD, PAD), jnp.float32)],
    )(col, row, cls_in)

    kept_boxes = jnp.stack(
        [vx1[0, :TOPK], vy1[0, :TOPK], vx2[0, :TOPK], vy2[0, :TOPK]], axis=1)
    out = jnp.concatenate(
        [kept_boxes, vsc[0, :TOPK, None], vbf[0, :TOPK, None]], axis=1)
    kept_cls = vcl[0, :TOPK]
    return out, kept_cls


# speedup vs baseline: 8.1202x; 1.1948x over previous
import jax
import jax.numpy as jnp
from jax.experimental import pallas as pl
from jax.experimental.pallas import tpu as pltpu

N = 5000
K = 80
IMG_H = 800.0
IMG_W = 1333.0
SCORE_THRESH = 0.05
NMS_THRESH = 0.5
PRE_NMS = 1000
TOPK = 100
PAD = 1024
NEG = -3.0e38


def _nms_body(col_ref, row_ref, cls_ref, ox1, oy1, ox2, oy2, osc, obf, ocls):
    c = col_ref[...]
    r = row_ref[...]
    off_c = c[:, 4:5]
    ax1 = c[:, 0:1] + off_c
    ay1 = c[:, 1:2] + off_c
    ax2 = c[:, 2:3] + off_c
    ay2 = c[:, 3:4] + off_c
    off_r = r[6:7, :]
    bx1 = r[0:1, :] + off_r
    by1 = r[1:2, :] + off_r
    bx2 = r[2:3, :] + off_r
    by2 = r[3:4, :] + off_r
    area_a = (c[:, 2:3] - c[:, 0:1]) * (c[:, 3:4] - c[:, 1:2])
    area_b = (r[2:3, :] - r[0:1, :]) * (r[3:4, :] - r[1:2, :])
    iw = jnp.clip(jnp.minimum(ax2, bx2) - jnp.maximum(ax1, bx1), 0.0, None)
    ih = jnp.clip(jnp.minimum(ay2, by2) - jnp.maximum(ay1, by1), 0.0, None)
    inter = iw * ih
    union = area_a + area_b - inter
    iou = jnp.where(union > 0.0, inter / union, 0.0)

    iota = jax.lax.broadcasted_iota(jnp.int32, (1, PAD), 1)
    row_i = jax.lax.broadcasted_iota(jnp.int32, (PAD, PAD), 0)
    col_i = jax.lax.broadcasted_iota(jnp.int32, (PAD, PAD), 1)
    sup_m = jnp.where((iou > NMS_THRESH) & (row_i < col_i), 1.0, 0.0)

    s_row = r[4:5, :]
    keep0 = (s_row > NEG).astype(jnp.float32)

    def cond(carry):
        prev, keep = carry
        return jnp.any(prev != keep)

    def body(carry):
        _, keep = carry
        sup = jnp.dot(keep, sup_m, preferred_element_type=jnp.float32)
        new = jnp.where(sup > 0.5, 0.0, keep0)
        return keep, new

    _, keep = jax.lax.while_loop(cond, body, (keep0 - 1.0, keep0))
    final0 = jnp.where(keep > 0.0, s_row, NEG)

    bf_row = r[5:6, :]
    cls_row = cls_ref[...]
    iota_t = jax.lax.broadcasted_iota(jnp.int32, (1, 128), 1)
    zf = jnp.zeros((1, 128), jnp.float32)
    zi = jnp.zeros((1, 128), jnp.int32)

    def ext(t, carry):
        final, vx1, vy1, vx2, vy2, vsc, vbf, vcl = carry
        m = jnp.max(final)
        idx = jnp.min(jnp.where(final == m, iota, PAD))
        sel = iota == idx
        valid = m > NEG

        def pick(row):
            return jnp.sum(jnp.where(sel, row, 0.0))

        px1 = jnp.where(valid, pick(r[0:1, :]), 0.0)
        py1 = jnp.where(valid, pick(r[1:2, :]), 0.0)
        px2 = jnp.where(valid, pick(r[2:3, :]), 0.0)
        py2 = jnp.where(valid, pick(r[3:4, :]), 0.0)
        psc = jnp.where(valid, m, 0.0)
        pbf = jnp.where(valid, pick(bf_row), 0.0)
        pcl = jnp.where(valid, jnp.sum(jnp.where(sel, cls_row, 0)), -1)

        tt = iota_t == t
        carry2 = (jnp.where(sel, NEG, final),
                  jnp.where(tt, px1, vx1), jnp.where(tt, py1, vy1),
                  jnp.where(tt, px2, vx2), jnp.where(tt, py2, vy2),
                  jnp.where(tt, psc, vsc), jnp.where(tt, pbf, vbf),
                  jnp.where(tt, pcl, vcl))
        return carry2

    final, vx1, vy1, vx2, vy2, vsc, vbf, vcl = jax.lax.fori_loop(
        0, TOPK, ext, (final0, zf, zf, zf, zf, zf, zf, zi))
    ox1[...] = vx1
    oy1[...] = vy1
    ox2[...] = vx2
    oy2[...] = vy2
    osc[...] = vsc
    obf[...] = vbf
    ocls[...] = vcl


def kernel(boxes, scores, scores_bf_multiply):
    x1 = jnp.clip(boxes[:, 0], 0.0, IMG_W)
    y1 = jnp.clip(boxes[:, 1], 0.0, IMG_H)
    x2 = jnp.clip(boxes[:, 2], 0.0, IMG_W)
    y2 = jnp.clip(boxes[:, 3], 0.0, IMG_H)
    boxes_c = jnp.stack([x1, y1, x2, y2], axis=1)

    cls_scores = scores[:, :-1].reshape(-1)
    bf_scores = scores_bf_multiply[:, :-1].reshape(-1)
    masked = jnp.where(cls_scores > SCORE_THRESH, cls_scores, -jnp.inf)
    top_scores, order = jax.lax.top_k(masked, PRE_NMS)
    box_idx = order // K
    cls_idx = order % K
    cand = boxes_c[box_idx]
    top_bf = bf_scores[order]
    maxv = jnp.max(boxes_c)
    off = (maxv + 1.0) * cls_idx.astype(jnp.float32)

    pad = PAD - PRE_NMS
    cand_p = jnp.pad(cand, ((0, pad), (0, 0)))
    sc_p = jnp.pad(jnp.where(jnp.isfinite(top_scores), top_scores, NEG),
                   (0, pad), constant_values=NEG)
    bf_p = jnp.pad(top_bf, (0, pad))
    off_p = jnp.pad(off, (0, pad))
    cls_p = jnp.pad(cls_idx, (0, pad))

    col = jnp.concatenate(
        [cand_p, off_p[:, None], jnp.zeros((PAD, 3), jnp.float32)], axis=1)
    row = jnp.stack([cand_p[:, 0], cand_p[:, 1], cand_p[:, 2], cand_p[:, 3],
                     sc_p, bf_p, off_p, jnp.zeros((PAD,), jnp.float32)],
                    axis=0)
    cls_in = cls_p[None, :].astype(jnp.int32)

    of = jax.ShapeDtypeStruct((1, 128), jnp.float32)
    oi = jax.ShapeDtypeStruct((1, 128), jnp.int32)
    vx1, vy1, vx2, vy2, vsc, vbf, vcl = pl.pallas_call(
        _nms_body,
        out_shape=[of, of, of, of, of, of, oi],
    )(col, row, cls_in)

    kept_boxes = jnp.stack(
        [vx1[0, :TOPK], vy1[0, :TOPK], vx2[0, :TOPK], vy2[0, :TOPK]], axis=1)
    out = jnp.concatenate(
        [kept_boxes, vsc[0, :TOPK, None], vbf[0, :TOPK, None]], axis=1)
    kept_cls = vcl[0, :TOPK]
    return out, kept_cls
